# Initial kernel scaffold; baseline (speedup 1.0000x reference)
import jax
import jax.numpy as jnp
from jax.experimental import pallas as pl


def kernel(x, embeddings):
    raise NotImplementedError("write your pallas kernel here")


# speedup vs baseline: 2.1875x; 2.1875x over previous
import jax
import jax.numpy as jnp
from jax.experimental import pallas as pl
from jax.experimental.pallas import tpu as pltpu

_N_EMB = 8192
_DIM = 256
_BETA = 0.25
_EPS = 1e-05
_DIVERSITY = 0.8
_TM = 256

def _bdot(a, b):
    return jnp.dot(a.astype(jnp.bfloat16), b.astype(jnp.bfloat16),
                   preferred_element_type=jnp.float32)


def _vq_body(x_ref, xt_ref, e_ref, et_ref,
             out_ref, loss_ref, perp_ref,
             esq_ref, colsum_ref, dwt_ref, sqerr_ref):
    i = pl.program_id(0)
    nsteps = pl.num_programs(0)
    m_total = nsteps * _TM

    @pl.when(i == 0)
    def _init():
        et0 = et_ref[...]
        esq_ref[...] = jnp.sum(et0 * et0, axis=0, keepdims=True)
        colsum_ref[...] = jnp.zeros_like(colsum_ref)
        dwt_ref[...] = jnp.zeros_like(dwt_ref)
        sqerr_ref[...] = jnp.zeros_like(sqerr_ref)

    x = x_ref[...]
    logits = 2.0 * _bdot(x, et_ref[...]) - esq_ref[...]
    mx = jnp.max(logits, axis=1, keepdims=True)
    p = jnp.exp(logits - mx)
    s = jnp.sum(p, axis=1, keepdims=True)
    enc = p / s

    q = _bdot(enc, e_ref[...])
    d = q - x
    out_ref[...] = x + d
    sqerr_ref[...] += jnp.sum(d * d, axis=(0, 1), keepdims=True)
    colsum_ref[...] += jnp.sum(enc, axis=0, keepdims=True)
    dwt_ref[...] += _bdot(xt_ref[...], enc)

    @pl.when(i == nsteps - 1)
    def _final():
        colsum = colsum_ref[...]
        avg_probs = colsum / m_total
        entropy = -jnp.sum(avg_probs * jnp.log(avg_probs + 1e-10),
                           axis=(0, 1), keepdims=True)
        usage = 0.01 * colsum
        up = usage / (jnp.sum(usage, axis=(0, 1), keepdims=True) + 1e-5)
        diversity = -jnp.sum(up * jnp.log(up + 1e-10),
                             axis=(0, 1), keepdims=True)
        ema_cs = 0.01 * colsum
        n = jnp.sum(ema_cs, axis=(0, 1), keepdims=True)
        cs = (ema_cs + _EPS) / (n + _N_EMB * _EPS) * n
        ema_wt = 0.01 * dwt_ref[...]
        r = jnp.sum(ema_wt * ema_wt, axis=0, keepdims=True)
        reg = jnp.sum(r / (cs * cs), axis=(0, 1), keepdims=True)
        mse = sqerr_ref[...] / (m_total * _DIM)
        loss_ref[...] = (mse + _BETA * mse + reg
                         + _DIVERSITY * (entropy + diversity))
        perp_ref[...] = jnp.exp(entropy)


def kernel(x, embeddings):
    xf = x.reshape(-1, _DIM)
    m = xf.shape[0]
    grid = (m // _TM,)
    out, loss, perp = pl.pallas_call(
        _vq_body,
        grid=grid,
        in_specs=[
            pl.BlockSpec((_TM, _DIM), lambda i: (i, 0)),
            pl.BlockSpec((_DIM, _TM), lambda i: (0, i)),
            pl.BlockSpec((_N_EMB, _DIM), lambda i: (0, 0)),
            pl.BlockSpec((_DIM, _N_EMB), lambda i: (0, 0)),
        ],
        out_specs=[
            pl.BlockSpec((_TM, _DIM), lambda i: (i, 0)),
            pl.BlockSpec((1, 1), lambda i: (0, 0)),
            pl.BlockSpec((1, 1), lambda i: (0, 0)),
        ],
        out_shape=[
            jax.ShapeDtypeStruct((m, _DIM), jnp.float32),
            jax.ShapeDtypeStruct((1, 1), jnp.float32),
            jax.ShapeDtypeStruct((1, 1), jnp.float32),
        ],
        scratch_shapes=[
            pltpu.VMEM((1, _N_EMB), jnp.float32),
            pltpu.VMEM((1, _N_EMB), jnp.float32),
            pltpu.VMEM((_DIM, _N_EMB), jnp.float32),
            pltpu.VMEM((1, 1), jnp.float32),
        ],
        compiler_params=pltpu.CompilerParams(
            dimension_semantics=("arbitrary",),
        ),
    )(xf, xf.T, embeddings, embeddings.T)
    return out.reshape(x.shape), loss[0, 0], perp[0, 0]
